# Initial kernel scaffold; baseline (speedup 1.0000x reference)
import jax
import jax.numpy as jnp
from jax.experimental import pallas as pl


def kernel(x, edge_index, W0, b0, W1, b1):
    raise NotImplementedError("write your pallas kernel here")


# speedup vs baseline: 6.6331x; 6.6331x over previous
import jax
import jax.numpy as jnp
from jax import lax
from jax.experimental import pallas as pl
from jax.experimental.pallas import tpu as pltpu
from jax.experimental.pallas import tpu_sc as plsc

N = 10000
E = 320000
D_IN = 128
U0 = 128
U1 = 64
NUM_ITER = 10
ALPHA = 0.1

NT = 16
NC = 2
CHUNK = 128
CPT = 168
E_PAD = NT * CPT * CHUNK
E_TOT = E + N
CPT_A = CPT // NC
IDXB = 8
NPT = N // NT
STRIDE = 640
NSR = NT * STRIDE
DUMMY = 128
AGG_R = NSR + DUMMY
SUB = 64
ROWS_B = 1000


def _zero_128(buf, rows):
    zero16 = jnp.zeros((16,), jnp.float32)

    def _z(k, _):
        for m in range(8):
            buf[k, pl.ds(m * 16, 16)] = zero16
        return 0
    lax.fori_loop(0, rows, _z, 0)


def _hist_body(row_hbm, out_hbm, row_v, ones_v, zbuf, agg_s):
    c = lax.axis_index("c")
    s = lax.axis_index("s")
    w = c * NT + s
    base = s * STRIDE
    one16 = jnp.ones((16,), jnp.float32)

    pltpu.sync_copy(row_hbm.at[w], row_v)

    def _fill_ones(k, _):
        for m in range(8):
            ones_v[k, pl.ds(m * 16, 16)] = one16
        return 0
    lax.fori_loop(0, CHUNK, _fill_ones, 0)
    _zero_128(zbuf, CHUNK)

    for u in range(STRIDE // CHUNK):
        pltpu.sync_copy(zbuf, agg_s.at[pl.ds(base + u * CHUNK, CHUNK)])

    @pl.when(s == NT - 1)
    def _zero_pad_rows():
        pltpu.sync_copy(zbuf, agg_s.at[pl.ds(NSR, DUMMY)])

    plsc.subcore_barrier()

    def _edge(j, _):
        pltpu.sync_copy(ones_v, agg_s.at[row_v.at[j]], add=True)
        return 0
    lax.fori_loop(0, CPT_A, _edge, 0)
    plsc.subcore_barrier()

    for u in range(STRIDE // CHUNK):
        pltpu.sync_copy(agg_s.at[pl.ds(base + u * CHUNK, CHUNK)],
                        out_hbm.at[c, pl.ds(base + u * CHUNK, CHUNK)])


def _hist_call(rowpad_a):
    mesh = plsc.VectorSubcoreMesh(core_axis_name="c", subcore_axis_name="s")
    return pl.kernel(
        _hist_body,
        mesh=mesh,
        out_type=jax.ShapeDtypeStruct((NC, NSR, CHUNK), jnp.float32),
        scratch_types=[
            pltpu.VMEM((CPT_A, CHUNK), jnp.int32),
            pltpu.VMEM((CHUNK, CHUNK), jnp.float32),
            pltpu.VMEM((CHUNK, CHUNK), jnp.float32),
            pltpu.VMEM_SHARED((AGG_R, CHUNK), jnp.float32),
        ],
    )(rowpad_a)


def _mlp_body(x_ref, w0_ref, b0_ref, w1_ref, b1_ref, deg_ref,
              y0_ref, dnv_ref):
    h = jnp.maximum(
        jnp.dot(x_ref[...], w0_ref[...], preferred_element_type=jnp.float32)
        + b0_ref[...], 0.0)
    h = jnp.dot(h, w1_ref[...], preferred_element_type=jnp.float32) + b1_ref[...]
    deg = deg_ref[...]
    dinv = lax.rsqrt(deg)
    y0_ref[...] = h * dinv
    dnv_ref[...] = jnp.broadcast_to(dinv * dinv, (ROWS_B, U1))


def _mlp_call(x, w0, b0, w1, b1, deg):
    g = N // ROWS_B
    return pl.pallas_call(
        _mlp_body,
        grid=(g,),
        in_specs=[
            pl.BlockSpec((ROWS_B, D_IN), lambda i: (i, 0)),
            pl.BlockSpec((D_IN, U0), lambda i: (0, 0)),
            pl.BlockSpec((1, U0), lambda i: (0, 0)),
            pl.BlockSpec((U0, U1), lambda i: (0, 0)),
            pl.BlockSpec((1, U1), lambda i: (0, 0)),
            pl.BlockSpec((ROWS_B, 1), lambda i: (i, 0)),
        ],
        out_specs=[
            pl.BlockSpec((ROWS_B, U1), lambda i: (i, 0)),
            pl.BlockSpec((ROWS_B, U1), lambda i: (i, 0)),
        ],
        out_shape=[
            jax.ShapeDtypeStruct((N, U1), jnp.float32),
            jax.ShapeDtypeStruct((N, U1), jnp.float32),
        ],
    )(x, w0, b0, w1, b1, deg)


def _prop_body(y0_hbm, dnv_hbm, col_hbm, row_hbm,
               ytab_hbm,
               col_b, row_b, gbuf, abuf, y0b, dnvb, agg_s):
    s = lax.axis_index("s")
    base = pl.multiple_of(s * STRIDE, 8)
    hbase = pl.multiple_of(s * (STRIDE // 2), 8)
    zero16 = jnp.zeros((16,), jnp.float32)

    _zero_128(gbuf, CHUNK)
    for u in range(STRIDE // CHUNK):
        pltpu.sync_copy(gbuf, agg_s.at[pl.ds(base + u * CHUNK, CHUNK)])

    @pl.when(s == NT - 1)
    def _zero_pad_rows():
        pltpu.sync_copy(gbuf, agg_s.at[pl.ds(NSR, DUMMY)])

    def _unpack_block(b, _):
        pltpu.sync_copy(
            y0_hbm.at[pl.ds(pl.multiple_of(hbase + b * (SUB // 2), 8),
                            SUB // 2)], y0b)

        def _un(n2, _):
            for h in range(4):
                abuf[2 * n2, pl.ds(h * 16, 16)] = y0b[n2, pl.ds(h * 16, 16)]
                abuf[2 * n2 + 1, pl.ds(h * 16, 16)] = \
                    y0b[n2, pl.ds(64 + h * 16, 16)]
                abuf[2 * n2, pl.ds(64 + h * 16, 16)] = zero16
                abuf[2 * n2 + 1, pl.ds(64 + h * 16, 16)] = zero16
            return 0
        lax.fori_loop(0, SUB // 2, _un, 0)
        pltpu.sync_copy(
            abuf, ytab_hbm.at[pl.ds(pl.multiple_of(base + b * SUB, 8), SUB)])
        return 0
    lax.fori_loop(0, STRIDE // SUB, _unpack_block, 0)
    plsc.subcore_barrier()

    for t in range(NUM_ITER):
        def _eblk(bi, _):
            pltpu.sync_copy(col_hbm.at[s, pl.ds(bi * IDXB, IDXB)], col_b)
            pltpu.sync_copy(row_hbm.at[s, pl.ds(bi * IDXB, IDXB)], row_b)
            for jj in range(IDXB):
                pltpu.sync_copy(ytab_hbm.at[col_b.at[jj]], gbuf)
                pltpu.sync_copy(gbuf, agg_s.at[row_b.at[jj]], add=True)
            return 0
        lax.fori_loop(0, CPT // IDXB, _eblk, 0)
        plsc.subcore_barrier()

        _zero_128(gbuf, CHUNK)

        def _nblk(b, _):
            off = pl.multiple_of(base + b * SUB, 8)
            off2 = pl.multiple_of(hbase + b * (SUB // 2), 8)
            pltpu.sync_copy(agg_s.at[pl.ds(off, SUB)], abuf)
            pltpu.sync_copy(gbuf.at[pl.ds(0, SUB)], agg_s.at[pl.ds(off, SUB)])
            pltpu.sync_copy(y0_hbm.at[pl.ds(off2, SUB // 2)], y0b)
            pltpu.sync_copy(dnv_hbm.at[pl.ds(off2, SUB // 2)], dnvb)

            def _nd(n2, _):
                for h in range(4):
                    sl = pl.ds(h * 16, 16)
                    sh = pl.ds(64 + h * 16, 16)
                    abuf[2 * n2, sl] = (1.0 - ALPHA) * dnvb[n2, sl] \
                        * abuf[2 * n2, sl] + ALPHA * y0b[n2, sl]
                    abuf[2 * n2 + 1, sl] = (1.0 - ALPHA) * dnvb[n2, sh] \
                        * abuf[2 * n2 + 1, sl] + ALPHA * y0b[n2, sh]
                return 0
            lax.fori_loop(0, SUB // 2, _nd, 0)
            pltpu.sync_copy(abuf, ytab_hbm.at[pl.ds(off, SUB)])
            return 0
        lax.fori_loop(0, STRIDE // SUB, _nblk, 0)
        plsc.subcore_barrier()


def _prop_call(y0p, dnvp, colpad_c, rowpad_c):
    mesh = plsc.VectorSubcoreMesh(core_axis_name="c", subcore_axis_name="s",
                                  num_cores=1)
    return pl.kernel(
        _prop_body,
        mesh=mesh,
        out_type=jax.ShapeDtypeStruct((NSR, CHUNK), jnp.float32),
        scratch_types=[
            pltpu.VMEM((IDXB, CHUNK), jnp.int32),
            pltpu.VMEM((IDXB, CHUNK), jnp.int32),
            pltpu.VMEM((CHUNK, CHUNK), jnp.float32),
            pltpu.VMEM((SUB, CHUNK), jnp.float32),
            pltpu.VMEM((SUB // 2, CHUNK), jnp.float32),
            pltpu.VMEM((SUB // 2, CHUNK), jnp.float32),
            pltpu.VMEM_SHARED((AGG_R, CHUNK), jnp.float32),
        ],
    )(y0p, dnvp, colpad_c, rowpad_c)


def _fin_body(y_ref, deg_ref, out_ref):
    out_ref[...] = y_ref[...] * jnp.sqrt(deg_ref[...])


def _fin_call(yfin, deg):
    g = N // ROWS_B
    return pl.pallas_call(
        _fin_body,
        grid=(g,),
        in_specs=[
            pl.BlockSpec((ROWS_B, U1), lambda i: (i, 0)),
            pl.BlockSpec((ROWS_B, 1), lambda i: (i, 0)),
        ],
        out_specs=pl.BlockSpec((ROWS_B, U1), lambda i: (i, 0)),
        out_shape=jax.ShapeDtypeStruct((N, U1), jnp.float32),
    )(yfin, deg)


def kernel(x, edge_index, W0, b0, W1, b1):
    ei = edge_index.astype(jnp.int32)

    def remap(v):
        return (v // NPT) * STRIDE + (v % NPT)

    rows = remap(ei[0])
    cols = remap(ei[1])
    self_idx = remap(jnp.arange(N, dtype=jnp.int32))
    npad = E_PAD - E_TOT
    pad_r = NSR + (jnp.arange(npad, dtype=jnp.int32) % DUMMY)
    pad_c = remap((jnp.arange(npad, dtype=jnp.int32) * 37) % N)
    rowpad = jnp.concatenate([rows, self_idx, pad_r])
    colpad = jnp.concatenate([cols, self_idx, pad_c])
    rowpad_a = rowpad.reshape(NC * NT, CPT_A, CHUNK)
    rowpad_c = rowpad.reshape(NT, CPT, CHUNK)
    colpad_c = colpad.reshape(NT, CPT, CHUNK)

    dega = _hist_call(rowpad_a)
    deg = dega[0, :, 0] + dega[1, :, 0]
    deg = deg.reshape(NT, STRIDE)[:, :NPT].reshape(N, 1)

    y0, dnv = _mlp_call(
        x, W0, b0.reshape(1, U0), W1, b1.reshape(1, U1), deg)

    def pack(v):
        v = jnp.pad(v.reshape(NT, NPT, U1), [(0, 0), (0, STRIDE - NPT), (0, 0)])
        return v.reshape(NSR // 2, 2 * U1)

    ytab = _prop_call(pack(y0), pack(dnv), colpad_c, rowpad_c)
    yfin = ytab.reshape(NT, STRIDE, CHUNK)[:, :NPT, :U1].reshape(N, U1)
    return _fin_call(yfin, deg)
